# baseline (speedup 1.0000x reference)
import functools

import jax
import jax.numpy as jnp
from jax import lax
from jax.experimental import pallas as pl
from jax.experimental.pallas import tpu as pltpu
from jax.experimental.pallas import tpu_sc as plsc

SEQ = 200
BATCH = 4096
DIM = 64
CLASSES = 100
VOCAB = 1000000

NW = 32
NBW = BATCH // NW
SCH = 2
NGRP = 2
NBUF = SCH * NGRP
NCHUNK = SEQ // SCH

XBLK = 8192
HBLK = XBLK // 2
NGRID = (VOCAB + XBLK - 1) // XBLK
VOCABP = NGRID * XBLK


def _tc_pack(tabT):
  def body(t_ref, o_ref):
    tr = t_ref[...].T
    o_ref[...] = jnp.concatenate([tr[:HBLK], tr[HBLK:]], axis=1)

  return pl.pallas_call(
      body,
      grid=(NGRID,),
      in_specs=[pl.BlockSpec((DIM, XBLK), lambda i: (0, i))],
      out_specs=pl.BlockSpec((HBLK, 2 * DIM), lambda i: (i, 0)),
      out_shape=jax.ShapeDtypeStruct((VOCABP // 2, 2 * DIM), jnp.float32),
  )(tabT)


def _sc_gather_max(x, table2):
  mesh = plsc.VectorSubcoreMesh(core_axis_name="c", subcore_axis_name="s")

  @functools.partial(
      pl.kernel,
      out_type=jax.ShapeDtypeStruct((BATCH, 128), jnp.float32),
      mesh=mesh,
      scratch_types=[
          pltpu.VMEM((SEQ, NBW), jnp.int32),
          pltpu.VMEM((SEQ, NBW), jnp.int32),
          pltpu.VMEM((NBUF, NBW, 128), jnp.float32),
          pltpu.VMEM((DIM, NBW), jnp.float32),
          pltpu.SemaphoreType.DMA,
          pltpu.SemaphoreType.DMA,
      ],
      compiler_params=pltpu.CompilerParams(needs_layout_passes=False),
  )
  def body(x_hbm, tab_hbm, m_hbm, idx_v, off_v, bufs, acc, sem0, sem1):
    wid = lax.axis_index("s") * 2 + lax.axis_index("c")
    base = wid * NBW

    pltpu.sync_copy(x_hbm.at[:, pl.ds(base, NBW)], idx_v)

    def prep_row(s, carry):
      for k in range(NBW // 16):
        v = idx_v[s, pl.ds(k * 16, 16)]
        loc = v & (XBLK - 1)
        idx_v[s, pl.ds(k * 16, 16)] = ((v >> 13) << 12) | (loc & (HBLK - 1))
        off_v[s, pl.ds(k * 16, 16)] = ((loc >> 12) & 1) << 6
      return carry

    lax.fori_loop(0, SEQ, prep_row, 0)

    def fire(c, g):
      sem = sem0 if g == 0 else sem1
      for j in range(SCH):
        pltpu.async_copy(
            tab_hbm.at[idx_v.at[c * SCH + j]], bufs.at[g * SCH + j], sem)

    def wait_group(g):
      sem = sem0 if g == 0 else sem1
      for j in range(SCH):
        pltpu.make_async_copy(
            tab_hbm.at[pl.ds(0, NBW)], bufs.at[g * SCH + j], sem).wait()

    neg_inf = jnp.full((16,), -jnp.inf, dtype=jnp.float32)

    def init_row(d, carry):
      for g in range(NBW // 16):
        acc[d, pl.ds(g * 16, 16)] = neg_inf
      return carry

    lax.fori_loop(0, DIM, init_row, 0)

    fire(0, 0)
    fire(1, 1)

    lane = lax.iota(jnp.int32, 16)

    def chunk_max(c, slot_base):
      def grp_body(g, carry):
        cols = g * 16 + lane
        for j in range(SCH):
          s = c * SCH + j
          off16 = off_v[s, pl.ds(g * 16, 16)]
          for d in range(DIM):
            val = plsc.load_gather(bufs.at[slot_base + j], [cols, off16 + d])
            acc[d, pl.ds(g * 16, 16)] = jnp.maximum(
                acc[d, pl.ds(g * 16, 16)], val)
        return carry

      lax.fori_loop(0, NBW // 16, grp_body, 0)

    def t_body(t, carry):
      for g in range(NGRP):
        c = NGRP * t + g
        wait_group(g)
        chunk_max(c, g * SCH)

        @pl.when(c + NGRP < NCHUNK)
        def _():
          fire(c + NGRP, g)
      return carry

    lax.fori_loop(0, NCHUNK // NGRP, t_body, 0)

    def out_row(r, carry):
      for k in range(DIM // 16):
        bufs[0, r, pl.ds(k * 16, 16)] = plsc.load_gather(
            acc, [k * 16 + lane, jnp.full((16,), r, jnp.int32)])
      return carry

    lax.fori_loop(0, NBW, out_row, 0)
    pltpu.sync_copy(bufs.at[0], m_hbm.at[pl.ds(base, NBW)])

  return body(x, table2)


def _tc_linear(m, w_pad, b_pad):
  def body(m_ref, w_ref, b_ref, o_ref):
    o_ref[...] = (
        jnp.dot(m_ref[...], w_ref[...], preferred_element_type=jnp.float32)
        + b_ref[...])

  return pl.pallas_call(
      body,
      grid=(8,),
      in_specs=[
          pl.BlockSpec((BATCH // 8, DIM), lambda i: (i, 0)),
          pl.BlockSpec((DIM, 128), lambda i: (0, 0)),
          pl.BlockSpec((1, 128), lambda i: (0, 0)),
      ],
      out_specs=pl.BlockSpec((BATCH // 8, 128), lambda i: (i, 0)),
      out_shape=jax.ShapeDtypeStruct((BATCH, 128), jnp.float32),
  )(m, w_pad, b_pad)


def kernel(x, emb_table, fc_w, fc_b):
  x = x.astype(jnp.int32)
  table2 = _tc_pack(emb_table.T)
  m = _sc_gather_max(x, table2)[:, :DIM]
  w_pad = jnp.zeros((DIM, 128), jnp.float32).at[:, :CLASSES].set(fc_w.T)
  b_pad = jnp.zeros((1, 128), jnp.float32).at[:, :CLASSES].set(fc_b[None, :])
  out = _tc_linear(m, w_pad, b_pad)
  return out[:, :CLASSES]


# speedup vs baseline: 3.1756x; 3.1756x over previous
import functools

import jax
import jax.numpy as jnp
from jax import lax
from jax.experimental import pallas as pl
from jax.experimental.pallas import tpu as pltpu
from jax.experimental.pallas import tpu_sc as plsc

SEQ = 200
BATCH = 4096
DIM = 64
CLASSES = 100

NW = 32
NBW = BATCH // NW
SCH = 2
NGRP = 2
NBUF = SCH * NGRP
NCHUNK = SEQ // SCH
VPR = DIM // 16
HVOCAB = 500000
VOCAB = 1000000
XBLK = 16384


NGRID = (VOCAB + XBLK - 1) // XBLK
VOCABP = NGRID * XBLK


def _tc_pack(tabT):
  def body(t_ref, o_ref):
    tr = t_ref[...].T
    o_ref[...] = jnp.concatenate([tr, tr], axis=1)

  return pl.pallas_call(
      body,
      grid=(NGRID,),
      in_specs=[pl.BlockSpec((DIM, XBLK), lambda i: (0, i))],
      out_specs=pl.BlockSpec((XBLK, 2 * DIM), lambda i: (i, 0)),
      out_shape=jax.ShapeDtypeStruct((VOCABP, 2 * DIM), jnp.float32),
  )(tabT)


def _sc_gather_max(x, table2):
  mesh = plsc.VectorSubcoreMesh(core_axis_name="c", subcore_axis_name="s")

  @functools.partial(
      pl.kernel,
      out_type=jax.ShapeDtypeStruct((BATCH, DIM), jnp.float32),
      mesh=mesh,
      scratch_types=[
          pltpu.VMEM((SEQ, NBW), jnp.int32),
          pltpu.VMEM((NBUF, NBW, 128), jnp.float32),
          pltpu.VMEM((NBW, DIM), jnp.float32),
          pltpu.SemaphoreType.DMA,
          pltpu.SemaphoreType.DMA,
      ],
  )
  def body(x_hbm, tab_hbm, m_hbm, idx_v, bufs, acc, sem0, sem1):
    wid = lax.axis_index("s") * 2 + lax.axis_index("c")
    base = wid * NBW

    pltpu.sync_copy(x_hbm.at[:, pl.ds(base, NBW)], idx_v)

    def fire(c, g):
      sem = sem0 if g == 0 else sem1
      for j in range(SCH):
        pltpu.async_copy(
            tab_hbm.at[idx_v.at[c * SCH + j]], bufs.at[g * SCH + j], sem)

    def wait_group(g):
      sem = sem0 if g == 0 else sem1
      for j in range(SCH):
        pltpu.make_async_copy(
            tab_hbm.at[pl.ds(0, NBW)], bufs.at[g * SCH + j], sem).wait()

    neg_inf = jnp.full((16,), -jnp.inf, dtype=jnp.float32)

    def init_row(r, carry):
      for k in range(VPR):
        acc[r, pl.ds(k * 16, 16)] = neg_inf
      return carry

    lax.fori_loop(0, NBW, init_row, 0)

    fire(0, 0)
    fire(1, 1)

    def chunk_max(slot_base):
      def row_body(r, carry):
        for k in range(VPR):
          v = acc[r, pl.ds(k * 16, 16)]
          for j in range(SCH):
            v = jnp.maximum(v, bufs[slot_base + j, r, pl.ds(k * 16, 16)])
          acc[r, pl.ds(k * 16, 16)] = v
        return carry

      lax.fori_loop(0, NBW, row_body, 0)

    def t_body(t, carry):
      for g in range(NGRP):
        c = NGRP * t + g
        wait_group(g)
        chunk_max(g * SCH)

        @pl.when(c + NGRP < NCHUNK)
        def _():
          fire(c + NGRP, g)
      return carry

    lax.fori_loop(0, NCHUNK // NGRP, t_body, 0)

    pltpu.sync_copy(acc, m_hbm.at[pl.ds(base, NBW)])

  return body(x, table2)


def _tc_linear(m, w_pad, b_pad):
  def body(m_ref, w_ref, b_ref, o_ref):
    o_ref[...] = (
        jnp.dot(m_ref[...], w_ref[...], preferred_element_type=jnp.float32)
        + b_ref[...])

  return pl.pallas_call(
      body,
      grid=(8,),
      in_specs=[
          pl.BlockSpec((BATCH // 8, DIM), lambda i: (i, 0)),
          pl.BlockSpec((DIM, 128), lambda i: (0, 0)),
          pl.BlockSpec((1, 128), lambda i: (0, 0)),
      ],
      out_specs=pl.BlockSpec((BATCH // 8, 128), lambda i: (i, 0)),
      out_shape=jax.ShapeDtypeStruct((BATCH, 128), jnp.float32),
  )(m, w_pad, b_pad)


def kernel(x, emb_table, fc_w, fc_b):
  x = x.astype(jnp.int32)
  table2 = _tc_pack(emb_table.T)
  m = _sc_gather_max(x, table2)
  w_pad = jnp.zeros((DIM, 128), jnp.float32).at[:, :CLASSES].set(fc_w.T)
  b_pad = jnp.zeros((1, 128), jnp.float32).at[:, :CLASSES].set(fc_b[None, :])
  out = _tc_linear(m, w_pad, b_pad)
  return out[:, :CLASSES]


# speedup vs baseline: 3.2455x; 1.0220x over previous
import functools

import jax
import jax.numpy as jnp
from jax import lax
from jax.experimental import pallas as pl
from jax.experimental.pallas import tpu as pltpu
from jax.experimental.pallas import tpu_sc as plsc

SEQ = 200
BATCH = 4096
DIM = 64
CLASSES = 100

NW = 32
NBW = BATCH // NW
SCH = 2
NGRP = 2
NBUF = SCH * NGRP
NCHUNK = SEQ // SCH
VPR = DIM // 16
HVOCAB = 500000
VOCAB = 1000000
XBLK = 24576


NGRID = (VOCAB + XBLK - 1) // XBLK
VOCABP = NGRID * XBLK


def _tc_pack(tabT):
  def body(t_ref, o_ref):
    tr = t_ref[...].T
    o_ref[...] = jnp.concatenate([tr, tr], axis=1)

  return pl.pallas_call(
      body,
      grid=(NGRID,),
      in_specs=[pl.BlockSpec((DIM, XBLK), lambda i: (0, i))],
      out_specs=pl.BlockSpec((XBLK, 2 * DIM), lambda i: (i, 0)),
      out_shape=jax.ShapeDtypeStruct((VOCABP, 2 * DIM), jnp.float32),
  )(tabT)


def _sc_gather_max(x, table2):
  mesh = plsc.VectorSubcoreMesh(core_axis_name="c", subcore_axis_name="s")

  @functools.partial(
      pl.kernel,
      out_type=jax.ShapeDtypeStruct((BATCH, DIM), jnp.float32),
      mesh=mesh,
      scratch_types=[
          pltpu.VMEM((SEQ, NBW), jnp.int32),
          pltpu.VMEM((NBUF, NBW, 128), jnp.float32),
          pltpu.VMEM((NBW, DIM), jnp.float32),
          pltpu.SemaphoreType.DMA,
          pltpu.SemaphoreType.DMA,
      ],
  )
  def body(x_hbm, tab_hbm, m_hbm, idx_v, bufs, acc, sem0, sem1):
    wid = lax.axis_index("s") * 2 + lax.axis_index("c")
    base = wid * NBW

    pltpu.sync_copy(x_hbm.at[:, pl.ds(base, NBW)], idx_v)

    def fire(c, g):
      sem = sem0 if g == 0 else sem1
      for j in range(SCH):
        pltpu.async_copy(
            tab_hbm.at[idx_v.at[c * SCH + j]], bufs.at[g * SCH + j], sem)

    def wait_group(g):
      sem = sem0 if g == 0 else sem1
      for j in range(SCH):
        pltpu.make_async_copy(
            tab_hbm.at[pl.ds(0, NBW)], bufs.at[g * SCH + j], sem).wait()

    neg_inf = jnp.full((16,), -jnp.inf, dtype=jnp.float32)

    def init_row(r, carry):
      for k in range(VPR):
        acc[r, pl.ds(k * 16, 16)] = neg_inf
      return carry

    lax.fori_loop(0, NBW, init_row, 0)

    fire(0, 0)
    fire(1, 1)

    def chunk_max(slot_base):
      def row_body(r, carry):
        for k in range(VPR):
          v = acc[r, pl.ds(k * 16, 16)]
          for j in range(SCH):
            v = jnp.maximum(v, bufs[slot_base + j, r, pl.ds(k * 16, 16)])
          acc[r, pl.ds(k * 16, 16)] = v
        return carry

      lax.fori_loop(0, NBW, row_body, 0)

    def t_body(t, carry):
      for g in range(NGRP):
        c = NGRP * t + g
        wait_group(g)
        chunk_max(g * SCH)

        @pl.when(c + NGRP < NCHUNK)
        def _():
          fire(c + NGRP, g)
      return carry

    lax.fori_loop(0, NCHUNK // NGRP, t_body, 0)

    pltpu.sync_copy(acc, m_hbm.at[pl.ds(base, NBW)])

  return body(x, table2)


def _tc_linear(m, w_pad, b_pad):
  def body(m_ref, w_ref, b_ref, o_ref):
    o_ref[...] = (
        jnp.dot(m_ref[...], w_ref[...], preferred_element_type=jnp.float32)
        + b_ref[...])

  return pl.pallas_call(
      body,
      grid=(8,),
      in_specs=[
          pl.BlockSpec((BATCH // 8, DIM), lambda i: (i, 0)),
          pl.BlockSpec((DIM, 128), lambda i: (0, 0)),
          pl.BlockSpec((1, 128), lambda i: (0, 0)),
      ],
      out_specs=pl.BlockSpec((BATCH // 8, 128), lambda i: (i, 0)),
      out_shape=jax.ShapeDtypeStruct((BATCH, 128), jnp.float32),
  )(m, w_pad, b_pad)


def kernel(x, emb_table, fc_w, fc_b):
  x = x.astype(jnp.int32)
  table2 = _tc_pack(emb_table.T)
  m = _sc_gather_max(x, table2)
  w_pad = jnp.zeros((DIM, 128), jnp.float32).at[:, :CLASSES].set(fc_w.T)
  b_pad = jnp.zeros((1, 128), jnp.float32).at[:, :CLASSES].set(fc_b[None, :])
  out = _tc_linear(m, w_pad, b_pad)
  return out[:, :CLASSES]


# speedup vs baseline: 3.6897x; 1.1369x over previous
import functools

import jax
import jax.numpy as jnp
from jax import lax
from jax.experimental import pallas as pl
from jax.experimental.pallas import tpu as pltpu
from jax.experimental.pallas import tpu_sc as plsc

SEQ = 200
BATCH = 4096
DIM = 64
CLASSES = 100
VOCAB = 1000000

NW = 32
NBW = BATCH // NW
SCH = 2
NGRP = 2
NBUF = SCH * NGRP
NCHUNK = SEQ // SCH

PBLK = 8192
XBLK = 3 * PBLK
NGRID = (VOCAB + XBLK - 1) // XBLK
VOCABP = NGRID * XBLK


def _tc_pack(tabT):
  def body(t_ref, o_ref):
    tr = t_ref[...].T
    parts = []
    for k in range(XBLK // PBLK):
      lo = tr[k * PBLK:k * PBLK + PBLK // 2]
      hi = tr[k * PBLK + PBLK // 2:(k + 1) * PBLK]
      parts.append(jnp.concatenate([lo, hi], axis=1))
    o_ref[...] = jnp.concatenate(parts, axis=0)

  return pl.pallas_call(
      body,
      grid=(NGRID,),
      in_specs=[pl.BlockSpec((DIM, XBLK), lambda i: (0, i))],
      out_specs=pl.BlockSpec((XBLK // 2, 2 * DIM), lambda i: (i, 0)),
      out_shape=jax.ShapeDtypeStruct((VOCABP // 2, 2 * DIM), jnp.float32),
  )(tabT)


def _sc_gather_max(x, table2):
  mesh = plsc.VectorSubcoreMesh(core_axis_name="c", subcore_axis_name="s")

  @functools.partial(
      pl.kernel,
      out_type=jax.ShapeDtypeStruct((BATCH, DIM), jnp.float32),
      mesh=mesh,
      scratch_types=[
          pltpu.VMEM((SEQ, NBW), jnp.int32),
          pltpu.VMEM((NCHUNK, NBW), jnp.int32),
          pltpu.VMEM((NBUF, NBW, 128), jnp.float32),
          pltpu.VMEM((NBW, DIM), jnp.float32),
          pltpu.SemaphoreType.DMA,
          pltpu.SemaphoreType.DMA,
      ],
      compiler_params=pltpu.CompilerParams(needs_layout_passes=False),
  )
  def body(x_hbm, tab_hbm, m_hbm, idx_v, off_v, bufs, acc, sem0, sem1):
    wid = lax.axis_index("s") * 2 + lax.axis_index("c")
    base = wid * NBW

    pltpu.sync_copy(x_hbm.at[:, pl.ds(base, NBW)], idx_v)

    def prep_row(s2, carry):
      for k in range(NBW // 16):
        v0 = idx_v[2 * s2, pl.ds(k * 16, 16)]
        v1 = idx_v[2 * s2 + 1, pl.ds(k * 16, 16)]
        idx_v[2 * s2, pl.ds(k * 16, 16)] = ((v0 >> 13) << 12) | (v0 & 4095)
        idx_v[2 * s2 + 1, pl.ds(k * 16, 16)] = (
            (v1 >> 13) << 12) | (v1 & 4095)
        off_v[s2, pl.ds(k * 16, 16)] = (
            ((v0 >> 12) & 1) | (((v1 >> 12) & 1) << 1))
      return carry

    lax.fori_loop(0, NCHUNK, prep_row, 0)

    def fire(c, g):
      sem = sem0 if g == 0 else sem1
      for j in range(SCH):
        pltpu.async_copy(
            tab_hbm.at[idx_v.at[c * SCH + j]], bufs.at[g * SCH + j], sem)

    def wait_group(g):
      sem = sem0 if g == 0 else sem1
      for j in range(SCH):
        pltpu.make_async_copy(
            tab_hbm.at[pl.ds(0, NBW)], bufs.at[g * SCH + j], sem).wait()

    neg_inf = jnp.full((16,), -jnp.inf, dtype=jnp.float32)

    def init_row(r, carry):
      for k in range(DIM // 16):
        acc[r, pl.ds(k * 16, 16)] = neg_inf
      return carry

    lax.fori_loop(0, NBW, init_row, 0)

    fire(0, 0)
    fire(1, 1)

    def chunk_max(c, slot_base):
      def row_body(r, carry):
        rb = jnp.full((16,), r, jnp.int32)
        cb = jnp.full((16,), c, jnp.int32)
        halves = plsc.load_gather(off_v, [cb, rb])
        vs = [acc[r, pl.ds(k * 16, 16)] for k in range(DIM // 16)]
        for j in range(SCH):
          msk = ((halves >> j) & 1) > 0
          for k in range(DIM // 16):
            lo = bufs[slot_base + j, r, pl.ds(k * 16, 16)]
            hi = bufs[slot_base + j, r, pl.ds(DIM + k * 16, 16)]
            vs[k] = jnp.maximum(vs[k], jnp.where(msk, hi, lo))
        for k in range(DIM // 16):
          acc[r, pl.ds(k * 16, 16)] = vs[k]
        return carry

      lax.fori_loop(0, NBW, row_body, 0)

    def t_body(t, carry):
      for g in range(NGRP):
        c = NGRP * t + g
        wait_group(g)
        chunk_max(c, g * SCH)

        @pl.when(c + NGRP < NCHUNK)
        def _():
          fire(c + NGRP, g)
      return carry

    lax.fori_loop(0, NCHUNK // NGRP, t_body, 0)

    pltpu.sync_copy(acc, m_hbm.at[pl.ds(base, NBW)])

  return body(x, table2)


def _tc_linear(m, w_pad, b_pad):
  def body(m_ref, w_ref, b_ref, o_ref):
    o_ref[...] = (
        jnp.dot(m_ref[...], w_ref[...], preferred_element_type=jnp.float32)
        + b_ref[...])

  return pl.pallas_call(
      body,
      grid=(8,),
      in_specs=[
          pl.BlockSpec((BATCH // 8, DIM), lambda i: (i, 0)),
          pl.BlockSpec((DIM, 128), lambda i: (0, 0)),
          pl.BlockSpec((1, 128), lambda i: (0, 0)),
      ],
      out_specs=pl.BlockSpec((BATCH // 8, 128), lambda i: (i, 0)),
      out_shape=jax.ShapeDtypeStruct((BATCH, 128), jnp.float32),
  )(m, w_pad, b_pad)


def kernel(x, emb_table, fc_w, fc_b):
  x = x.astype(jnp.int32)
  table2 = _tc_pack(emb_table.T)
  m = _sc_gather_max(x, table2)
  w_pad = jnp.zeros((DIM, 128), jnp.float32).at[:, :CLASSES].set(fc_w.T)
  b_pad = jnp.zeros((1, 128), jnp.float32).at[:, :CLASSES].set(fc_b[None, :])
  out = _tc_linear(m, w_pad, b_pad)
  return out[:, :CLASSES]
